# Initial kernel scaffold; baseline (speedup 1.0000x reference)
import jax
import jax.numpy as jnp
from jax.experimental import pallas as pl


def kernel(indices, W, b, ln_gamma, ln_beta, theta, base_re, base_im):
    raise NotImplementedError("write your pallas kernel here")


# speedup vs baseline: 12.4962x; 12.4962x over previous
import functools
import math

import jax
import jax.numpy as jnp
from jax import lax
from jax.experimental import pallas as pl
from jax.experimental.pallas import tpu as pltpu
from jax.experimental.pallas import tpu_sc as plsc

EMBED = 64
ROWS = 95
ROWS_PAD = 96
NC = 2
NS = 16
NW = NC * NS
CHUNK = 128


def _table_body(re_ref, im_ref, wt_ref, b_ref, g_ref, bt_ref, out_ref):
    acc = jnp.zeros((ROWS_PAD, EMBED), jnp.float32)
    for s in range(4):
        r = re_ref[s]
        i = im_ref[s]
        acc = acc + r * r + i * i
    nrm = jnp.sqrt(jnp.sum(acc, axis=1, keepdims=True))
    emb = jnp.sqrt(acc / ((nrm + 1e-8) ** 2) + 1e-12)
    out = jnp.dot(emb, wt_ref[...], preferred_element_type=jnp.float32)
    out = out + b_ref[...]
    mu = jnp.mean(out, axis=1, keepdims=True)
    xc = out - mu
    var = jnp.mean(xc * xc, axis=1, keepdims=True)
    out_ref[...] = xc * lax.rsqrt(var + 1e-5) * g_ref[...] + bt_ref[...]


@functools.lru_cache(maxsize=None)
def _make_gather(BL: int):
    per_w = BL // NW
    T = per_w // CHUNK
    mesh = plsc.VectorSubcoreMesh(core_axis_name="c", subcore_axis_name="s")

    @functools.partial(
        pl.kernel,
        mesh=mesh,
        out_type=jax.ShapeDtypeStruct((BL, EMBED), jnp.float32),
        scratch_types=[
            pltpu.VMEM((T, CHUNK), jnp.int32),
            pltpu.VMEM((CHUNK, EMBED), jnp.float32),
            pltpu.VMEM((CHUNK, EMBED), jnp.float32),
            pltpu.SemaphoreType.DMA,
            pltpu.SemaphoreType.DMA,
        ],
        compiler_params=pltpu.CompilerParams(use_tc_tiling_on_sc=False),
    )
    def gather_kernel(table_hbm, idx_hbm, out_hbm, idx_v, buf_a, buf_b, sem_a, sem_b):
        wid = lax.axis_index("s") * NC + lax.axis_index("c")
        base = wid * per_w
        pltpu.sync_copy(idx_hbm.at[wid], idx_v)

        def body(t, carry):
            pltpu.async_copy(table_hbm.at[idx_v.at[t]], buf_a, sem_a).wait()
            pltpu.sync_copy(buf_a, out_hbm.at[pl.ds(base + t * CHUNK, CHUNK)])
            return carry

        lax.fori_loop(0, T, body, 0)

    return gather_kernel


def kernel(indices, W, b, ln_gamma, ln_beta, theta, base_re, base_im):
    Bq, L = indices.shape
    BL = Bq * L
    re_t = jnp.pad(jnp.transpose(base_re, (2, 0, 1)),
                   ((0, 0), (0, ROWS_PAD - ROWS), (0, 0)))
    im_t = jnp.pad(jnp.transpose(base_im, (2, 0, 1)),
                   ((0, 0), (0, ROWS_PAD - ROWS), (0, 0)))
    table = pl.pallas_call(
        _table_body,
        out_shape=jax.ShapeDtypeStruct((ROWS_PAD, EMBED), jnp.float32),
    )(re_t, im_t, W.T, b.reshape(1, EMBED),
      ln_gamma.reshape(1, EMBED), ln_beta.reshape(1, EMBED))

    idx2 = indices.reshape(-1).astype(jnp.int32).reshape(NW, BL // (NW * CHUNK), CHUNK)
    flat = _make_gather(BL)(table, idx2)
    return flat.reshape(Bq, L, EMBED)


# speedup vs baseline: 12.5605x; 1.0051x over previous
import functools
import math

import jax
import jax.numpy as jnp
from jax import lax
from jax.experimental import pallas as pl
from jax.experimental.pallas import tpu as pltpu
from jax.experimental.pallas import tpu_sc as plsc

EMBED = 64
ROWS = 95
ROWS_PAD = 96
NC = 2
NS = 16
NW = NC * NS
CHUNK = 128


def _table_body(re_ref, im_ref, wt_ref, b_ref, g_ref, bt_ref, out_ref):
    acc = jnp.zeros((ROWS_PAD, EMBED), jnp.float32)
    for s in range(4):
        r = re_ref[s]
        i = im_ref[s]
        acc = acc + r * r + i * i
    nrm = jnp.sqrt(jnp.sum(acc, axis=1, keepdims=True))
    emb = jnp.sqrt(acc / ((nrm + 1e-8) ** 2) + 1e-12)
    out = jnp.dot(emb, wt_ref[...], preferred_element_type=jnp.float32)
    out = out + b_ref[...]
    mu = jnp.mean(out, axis=1, keepdims=True)
    xc = out - mu
    var = jnp.mean(xc * xc, axis=1, keepdims=True)
    out_ref[...] = xc * lax.rsqrt(var + 1e-5) * g_ref[...] + bt_ref[...]


@functools.lru_cache(maxsize=None)
def _make_gather(BL: int):
    per_w = BL // NW
    T = per_w // CHUNK
    mesh = plsc.VectorSubcoreMesh(core_axis_name="c", subcore_axis_name="s")

    NBUF = 5
    assert T % NBUF == 0
    scratch = [pltpu.VMEM((T, CHUNK), jnp.int32)]
    scratch += [pltpu.VMEM((CHUNK, EMBED), jnp.float32) for _ in range(NBUF)]
    scratch += [pltpu.SemaphoreType.DMA for _ in range(NBUF)]

    @functools.partial(
        pl.kernel,
        mesh=mesh,
        out_type=jax.ShapeDtypeStruct((BL, EMBED), jnp.float32),
        scratch_types=scratch,
        compiler_params=pltpu.CompilerParams(use_tc_tiling_on_sc=False),
    )
    def gather_kernel(table_hbm, idx_hbm, out_hbm, idx_v, *bufsem):
        bufs = bufsem[:NBUF]
        sems = bufsem[NBUF:]
        wid = lax.axis_index("s") * NC + lax.axis_index("c")
        base = wid * per_w
        pltpu.sync_copy(idx_hbm.at[wid], idx_v)
        for k in range(NBUF):
            pltpu.async_copy(table_hbm.at[idx_v.at[k]], bufs[k], sems[k])

        def body(i, carry):
            g = i * NBUF
            for k in range(NBUF):
                t = g + k
                pltpu.make_async_copy(
                    table_hbm.at[idx_v.at[t]], bufs[k], sems[k]).wait()
                pltpu.sync_copy(bufs[k], out_hbm.at[pl.ds(base + t * CHUNK, CHUNK)])

                @pl.when(t + NBUF < T)
                def _():
                    pltpu.async_copy(
                        table_hbm.at[idx_v.at[t + NBUF]], bufs[k], sems[k])

            return carry

        lax.fori_loop(0, T // NBUF, body, 0)

    return gather_kernel


def kernel(indices, W, b, ln_gamma, ln_beta, theta, base_re, base_im):
    Bq, L = indices.shape
    BL = Bq * L
    re_t = jnp.pad(jnp.transpose(base_re, (2, 0, 1)),
                   ((0, 0), (0, ROWS_PAD - ROWS), (0, 0)))
    im_t = jnp.pad(jnp.transpose(base_im, (2, 0, 1)),
                   ((0, 0), (0, ROWS_PAD - ROWS), (0, 0)))
    table = pl.pallas_call(
        _table_body,
        out_shape=jax.ShapeDtypeStruct((ROWS_PAD, EMBED), jnp.float32),
    )(re_t, im_t, W.T, b.reshape(1, EMBED),
      ln_gamma.reshape(1, EMBED), ln_beta.reshape(1, EMBED))

    idx2 = indices.reshape(-1).astype(jnp.int32).reshape(NW, BL // (NW * CHUNK), CHUNK)
    flat = _make_gather(BL)(table, idx2)
    return flat.reshape(Bq, L, EMBED)


# speedup vs baseline: 24.4895x; 1.9497x over previous
import functools
import math

import jax
import jax.numpy as jnp
from jax import lax
from jax.experimental import pallas as pl
from jax.experimental.pallas import tpu as pltpu
from jax.experimental.pallas import tpu_sc as plsc

EMBED = 64
ROWS = 95
ROWS_PAD = 96
NC = 2
NS = 16
NW = NC * NS
CHUNK = 128


def _table_body(re_ref, im_ref, wt_ref, b_ref, g_ref, bt_ref, out_ref):
    acc = jnp.zeros((ROWS_PAD, EMBED), jnp.float32)
    for s in range(4):
        r = re_ref[s]
        i = im_ref[s]
        acc = acc + r * r + i * i
    nrm = jnp.sqrt(jnp.sum(acc, axis=1, keepdims=True))
    emb = jnp.sqrt(acc / ((nrm + 1e-8) ** 2) + 1e-12)
    out = jnp.dot(emb, wt_ref[...], preferred_element_type=jnp.float32)
    out = out + b_ref[...]
    mu = jnp.mean(out, axis=1, keepdims=True)
    xc = out - mu
    var = jnp.mean(xc * xc, axis=1, keepdims=True)
    out_ref[...] = xc * lax.rsqrt(var + 1e-5) * g_ref[...] + bt_ref[...]


@functools.lru_cache(maxsize=None)
def _make_gather(BL: int):
    per_w = BL // NW
    T = per_w // CHUNK
    mesh = plsc.VectorSubcoreMesh(core_axis_name="c", subcore_axis_name="s")

    NBUF = 5
    assert T % NBUF == 0
    scratch = [pltpu.VMEM((T, CHUNK), jnp.int32)]
    scratch += [pltpu.VMEM((CHUNK, EMBED), jnp.float32) for _ in range(NBUF)]
    scratch += [pltpu.SemaphoreType.DMA for _ in range(NBUF)]
    scratch += [pltpu.VMEM_SHARED((ROWS_PAD, EMBED), jnp.float32)]

    @functools.partial(
        pl.kernel,
        mesh=mesh,
        out_type=jax.ShapeDtypeStruct((BL, EMBED), jnp.float32),
        scratch_types=scratch,
        compiler_params=pltpu.CompilerParams(use_tc_tiling_on_sc=False),
    )
    def gather_kernel(table_hbm, idx_hbm, out_hbm, idx_v, *bufsem):
        bufs = bufsem[:NBUF]
        sems = bufsem[NBUF:NBUF * 2]
        tab_sh = bufsem[NBUF * 2]
        wid = lax.axis_index("s") * NC + lax.axis_index("c")
        base = wid * per_w

        @pl.when(lax.axis_index("s") == 0)
        def _():
            pltpu.sync_copy(table_hbm, tab_sh)

        pltpu.sync_copy(idx_hbm.at[wid], idx_v)
        plsc.subcore_barrier()
        for k in range(NBUF):
            pltpu.async_copy(tab_sh.at[idx_v.at[k]], bufs[k], sems[k])

        def body(i, carry):
            g = i * NBUF
            for k in range(NBUF):
                t = g + k
                pltpu.make_async_copy(
                    tab_sh.at[idx_v.at[t]], bufs[k], sems[k]).wait()
                pltpu.sync_copy(bufs[k], out_hbm.at[pl.ds(base + t * CHUNK, CHUNK)])

                @pl.when(t + NBUF < T)
                def _():
                    pltpu.async_copy(
                        tab_sh.at[idx_v.at[t + NBUF]], bufs[k], sems[k])

            return carry

        lax.fori_loop(0, T // NBUF, body, 0)

    return gather_kernel


def kernel(indices, W, b, ln_gamma, ln_beta, theta, base_re, base_im):
    Bq, L = indices.shape
    BL = Bq * L
    re_t = jnp.pad(jnp.transpose(base_re, (2, 0, 1)),
                   ((0, 0), (0, ROWS_PAD - ROWS), (0, 0)))
    im_t = jnp.pad(jnp.transpose(base_im, (2, 0, 1)),
                   ((0, 0), (0, ROWS_PAD - ROWS), (0, 0)))
    table = pl.pallas_call(
        _table_body,
        out_shape=jax.ShapeDtypeStruct((ROWS_PAD, EMBED), jnp.float32),
    )(re_t, im_t, W.T, b.reshape(1, EMBED),
      ln_gamma.reshape(1, EMBED), ln_beta.reshape(1, EMBED))

    idx2 = indices.reshape(-1).astype(jnp.int32).reshape(NW, BL // (NW * CHUNK), CHUNK)
    flat = _make_gather(BL)(table, idx2)
    return flat.reshape(Bq, L, EMBED)
